# baseline (speedup 1.0000x reference)
import functools
import jax
import jax.numpy as jnp
from jax import lax
from jax.experimental import pallas as pl
from jax.experimental.pallas import tpu as pltpu
from jax.experimental.pallas import tpu_sc as plsc

H = 128
NV = 16
F = 1000
FPAD = 1152
FOUT = 1024
B = 6400
W = 128
XC = H + 4
MIN_FRAG = 2.0
TEMP = 1.0
SLR = 0.5
VW = 0.1
CF = 0.05
DELTA = 0.1
NPAIR = F * (F - 1) // 2
HI = lax.Precision.DEFAULT

NW = 32
CG = 12
CROWS = CG * 16


def _silu(x):
    return x * jax.nn.sigmoid(x)


def _body_a(ids_ref, ss_ref, sl_ref,
            W1_ref, b1_ref, g_ref, bln_ref, W2_ref, b2_ref,
            out_ref, acc_ref):
    i = pl.program_id(0)
    nsteps = pl.num_programs(0)

    @pl.when(i == 0)
    def _init():
        acc_ref[...] = jnp.zeros_like(acc_ref)

    sc = ss_ref[...] * SLR + sl_ref[...] * (1.0 - SLR)
    z = jnp.dot(sc, W1_ref[...], preferred_element_type=jnp.float32,
                precision=HI) + b1_ref[0, :]
    mu = jnp.mean(z, axis=-1, keepdims=True)
    var = jnp.mean((z - mu) ** 2, axis=-1, keepdims=True)
    zn = (z - mu) / jnp.sqrt(var + 1e-5) * g_ref[0, :] + bln_ref[0, :]
    h = _silu(zn)
    sp = jnp.dot(h, W2_ref[...], preferred_element_type=jnp.float32,
                 precision=HI) + b2_ref[0, :]
    sq = jnp.sum(sp * sp, axis=1, keepdims=True)

    one = jnp.ones_like(sq)
    x = jnp.concatenate([sp, sq, one, jnp.zeros((B, 2), jnp.float32)], axis=1)

    idc = ids_ref[0]
    m8 = (jnp.min(idc) // 8) * 8
    nwin = (jnp.max(idc) - m8) // W + 1

    def win(w, _):
        ws = m8 + w * W
        lane = lax.broadcasted_iota(jnp.int32, (B, W), 1) + ws
        oh = (idc == lane).astype(jnp.float32)
        part = lax.dot_general(oh, x, (((0,), (0,)), ((), ())),
                               preferred_element_type=jnp.float32,
                               precision=HI)
        acc_ref[pl.ds(ws, W), :] += part
        return 0

    lax.fori_loop(0, nwin, win, 0)

    @pl.when(i == nsteps - 1)
    def _fin():
        out_ref[...] = acc_ref[:FOUT, :]


def _sqrt16(s):
    i = lax.bitcast_convert_type(s, jnp.int32)
    i = 0x5F3759DF - lax.shift_right_logical(i, 1)
    y = lax.bitcast_convert_type(i, jnp.float32)
    for _ in range(3):
        y = y * (1.5 - 0.5 * s * y * y)
    return s * y


def _body_b(vs_hbm, vl_hbm, ids_hbm, out_hbm, bufa, bufb, bufi, tab):
    nc = 2
    wid = lax.axis_index("s") * nc + lax.axis_index("c")
    start_g = wid * 313 - jnp.maximum(wid - 16, 0)
    ngroups = 313 - (wid >= 16).astype(jnp.int32)
    nch = (ngroups + CG - 1) // CG

    zeros16 = jnp.zeros((16,), jnp.float32)
    for j in range(2048 // 16):
        tab[pl.ds(j * 16, 16)] = zeros16

    lanes = lax.broadcasted_iota(jnp.int32, (16,), 0)

    def chunk(c, _):
        p0 = (start_g + c * CG) * 16
        pltpu.sync_copy(vs_hbm.at[pl.ds(p0 * 48, CROWS * 48)], bufa)
        pltpu.sync_copy(vl_hbm.at[pl.ds(p0 * 48, CROWS * 48)], bufb)
        pltpu.sync_copy(ids_hbm.at[pl.ds(p0, CROWS)], bufi)
        gmax = jnp.minimum(CG, ngroups - c * CG)

        def group(gl, _):
            base = gl * 768 + lanes * 48
            vacc = jnp.zeros((16,), jnp.float32)
            for v in range(NV):
                s = jnp.zeros((16,), jnp.float32)
                for k in range(3):
                    idx = base + (3 * v + k)
                    a = plsc.load_gather(bufa, [idx])
                    b = plsc.load_gather(bufb, [idx])
                    q = a + b
                    s = s + q * q
                vacc = vacc + _sqrt16(s)
            vmf = vacc * (0.5 / NV)
            ids16 = bufi[pl.ds(gl * 16, 16)]
            plsc.addupdate_scatter(tab, [ids16], vmf)
            plsc.addupdate_scatter(tab, [ids16 + 1024], vmf * vmf)
            return 0

        lax.fori_loop(0, gmax, group, 0)
        return 0

    lax.fori_loop(0, nch, chunk, 0)
    pltpu.sync_copy(tab, out_hbm.at[wid])


def _body_c(acc_ref, scp_ref, Wv_ref, Wt1_ref, bt1_ref, Wt2_ref, bt2_ref,
            out_ref):
    acc = acc_ref[...]
    s1 = acc[:, :H]
    s2 = acc[:, H:H + 1]
    cnt = acc[:, H + 1:H + 2]
    cnt1 = jnp.maximum(cnt, 1.0)

    svrow = scp_ref[0, :, :]
    for k in range(1, NW):
        svrow = svrow + scp_ref[k, :, :]
    svt = jnp.transpose(svrow)
    sv1 = svt[:FOUT, 0:1]
    sv2 = svt[:FOUT, 1:2]

    fmask = lax.broadcasted_iota(jnp.int32, (FOUT, 1), 0) < F
    valid = (cnt >= MIN_FRAG) & fmask
    vc = jnp.sum(valid.astype(jnp.float32))
    inv_vc = 1.0 / jnp.maximum(vc, 1.0)

    gf = s1 / cnt1
    m2 = jnp.sum(s1 * s1, axis=1, keepdims=True)
    varf = jnp.maximum((s2 - m2 / cnt1) / cnt1, 0.0)
    sqv = jnp.sqrt(varf + 1e-8)
    intra = jnp.where(vc > 0.0,
                      jnp.sum(jnp.where(valid, sqv, 0.0)) * inv_vc, 0.0)

    cw = jnp.sum(Wv_ref[...], axis=0, keepdims=True)
    s_w = jnp.sum(cw * cw)
    varv = jnp.maximum((sv2 - sv1 * sv1 / cnt1) / cnt1, 0.0) * s_w
    sqvv = jnp.sqrt(varv + 1e-8)
    vloss = jnp.where(vc > 0.0,
                      jnp.sum(jnp.where(valid, sqvv, 0.0)) * inv_vc, 0.0)

    u = _silu(jnp.dot(gf, Wt1_ref[...], preferred_element_type=jnp.float32,
                      precision=HI) + bt1_ref[0, :])
    tl = jnp.sum(u * Wt2_ref[0, :][None, :], axis=1,
                 keepdims=True) + bt2_ref[0, 0]
    t = jnp.clip(jax.nn.sigmoid(tl), 0.2, 0.8)
    avg_t = jnp.sum(jnp.where(fmask, t, 0.0)) * (1.0 / F)

    nrm = jnp.sqrt(jnp.sum(gf * gf, axis=1, keepdims=True))
    nf = gf / jnp.maximum(nrm, 1e-12)
    sim = lax.dot_general(nf, nf, (((1,), (1,)), ((), ())),
                          preferred_element_type=jnp.float32,
                          precision=HI) * (1.0 / TEMP)
    diff = sim - avg_t
    ad = jnp.abs(diff)
    hub = jnp.where(ad <= DELTA, 0.5 * diff * diff,
                    DELTA * (ad - 0.5 * DELTA))
    ri = lax.broadcasted_iota(jnp.int32, (FOUT, FOUT), 0)
    rj = lax.broadcasted_iota(jnp.int32, (FOUT, FOUT), 1)
    pmask = (ri < F) & (rj < F) & (ri != rj)
    gsl = jnp.sum(jnp.where(pmask, hub, 0.0)) * (0.5 / NPAIR)

    total = 0.3 * intra + 0.7 * gsl + VW * vloss
    out_ref[...] = jnp.full((1, 1), CF * total, dtype=jnp.float32)


def kernel(scalar_short, scalar_long, vector_short, vector_long, fragment_ids,
           W1, b1, ln_g, ln_b, W2, b2, Wv, Wt1, bt1, Wt2, bt2):
    n = scalar_short.shape[0]
    g = n // B
    ids3 = fragment_ids.astype(jnp.int32).reshape(g, B, 1)
    ids1 = fragment_ids.astype(jnp.int32)
    vs2 = vector_short.reshape(n, 3 * NV)
    vl2 = vector_long.reshape(n, 3 * NV)
    row = lambda v: v.reshape(1, -1)

    grid_spec = pltpu.PrefetchScalarGridSpec(
        num_scalar_prefetch=0,
        grid=(g,),
        in_specs=[
            pl.BlockSpec((1, B, 1), lambda i: (i, 0, 0)),
            pl.BlockSpec((B, H), lambda i: (i, 0)),
            pl.BlockSpec((B, H), lambda i: (i, 0)),
            pl.BlockSpec((H, H), lambda i: (0, 0)),
            pl.BlockSpec((1, H), lambda i: (0, 0)),
            pl.BlockSpec((1, H), lambda i: (0, 0)),
            pl.BlockSpec((1, H), lambda i: (0, 0)),
            pl.BlockSpec((H, H), lambda i: (0, 0)),
            pl.BlockSpec((1, H), lambda i: (0, 0)),
        ],
        out_specs=pl.BlockSpec((FOUT, XC), lambda i: (0, 0)),
        scratch_shapes=[pltpu.VMEM((FPAD, XC), jnp.float32)],
    )
    acc = pl.pallas_call(
        _body_a,
        grid_spec=grid_spec,
        out_shape=jax.ShapeDtypeStruct((FOUT, XC), jnp.float32),
        compiler_params=pltpu.CompilerParams(
            dimension_semantics=("arbitrary",)),
    )(ids3, scalar_short, scalar_long,
      W1, row(b1), row(ln_g), row(ln_b), W2, row(b2))

    mesh = plsc.VectorSubcoreMesh(core_axis_name="c", subcore_axis_name="s")
    scp = pl.kernel(
        _body_b,
        mesh=mesh,
        compiler_params=pltpu.CompilerParams(needs_layout_passes=False),
        out_type=jax.ShapeDtypeStruct((NW, 2048), jnp.float32),
        scratch_types=[
            pltpu.VMEM((CROWS * 3 * NV,), jnp.float32),
            pltpu.VMEM((CROWS * 3 * NV,), jnp.float32),
            pltpu.VMEM((CROWS,), jnp.int32),
            pltpu.VMEM((2048,), jnp.float32),
        ],
    )(vs2.reshape(-1), vl2.reshape(-1), ids1)

    scp3 = scp.reshape(NW, 2, FOUT)

    out = pl.pallas_call(
        _body_c,
        grid=(1,),
        in_specs=[
            pl.BlockSpec((FOUT, XC), lambda i: (0, 0)),
            pl.BlockSpec((NW, 2, FOUT), lambda i: (0, 0, 0)),
            pl.BlockSpec((H, H), lambda i: (0, 0)),
            pl.BlockSpec((H, 32), lambda i: (0, 0)),
            pl.BlockSpec((1, 32), lambda i: (0, 0)),
            pl.BlockSpec((1, 32), lambda i: (0, 0)),
            pl.BlockSpec((1, 1), lambda i: (0, 0)),
        ],
        out_specs=pl.BlockSpec((1, 1), lambda i: (0, 0)),
        out_shape=jax.ShapeDtypeStruct((1, 1), jnp.float32),
    )(acc, scp3, Wv, Wt1, row(bt1), row(Wt2), row(bt2))
    return out[0, 0]
